# baseline (speedup 1.0000x reference)
import functools

import jax
import jax.numpy as jnp
from jax import lax
from jax.experimental import pallas as pl
from jax.experimental.pallas import tpu as pltpu
from jax.experimental.pallas import tpu_sc as plsc

_LANES = 16


def _make_tag_kernel(n_workers, n_chunks, chans, n_parts, hw):
    mesh = plsc.VectorSubcoreMesh(core_axis_name="c", subcore_axis_name="s")

    @functools.partial(
        pl.kernel,
        mesh=mesh,
        out_type=jax.ShapeDtypeStruct((n_workers, _LANES), jnp.float32),
        scratch_types=[
            pltpu.VMEM((n_chunks, 128), jnp.int32),
            pltpu.VMEM((2 * n_chunks, 128), jnp.int32),
            pltpu.VMEM((2 * n_chunks, 128), jnp.float32),
            pltpu.VMEM((n_chunks, 128), jnp.float32),
            pltpu.VMEM((n_chunks, 128), jnp.float32),
            pltpu.VMEM((_LANES,), jnp.float32),
            pltpu.SemaphoreType.DMA,
        ],
    )
    def tag_kernel(preds_flat, kp_idx, gt, vis, out,
                   idx_v, addr_v, vals_v, gt_v, vis_v, acc_v, sem):
        wid = lax.axis_index("s") * 2 + lax.axis_index("c")
        b = wid // 2
        pltpu.sync_copy(kp_idx.at[b], idx_v)
        pltpu.sync_copy(gt.at[b], gt_v)
        pltpu.sync_copy(vis.at[b], vis_v)
        for t in range(2):
            j = wid * 2 + t
            base = (j * chans + n_parts) * hw
            for c in range(n_chunks):
                for i in range(128 // _LANES):
                    sl = pl.ds(i * _LANES, _LANES)
                    addr_v[t * n_chunks + c, sl] = idx_v[c, sl] + base
        copies = [
            pltpu.async_copy(preds_flat.at[addr_v.at[r]], vals_v.at[r], sem)
            for r in range(2 * n_chunks)
        ]
        for cp in copies:
            cp.wait()
        acc = jnp.zeros((_LANES,), jnp.float32)
        for t in range(2):
            for c in range(n_chunks):
                for i in range(128 // _LANES):
                    sl = pl.ds(i * _LANES, _LANES)
                    d = vals_v[t * n_chunks + c, sl] - gt_v[c, sl]
                    acc = acc + d * d * vis_v[c, sl]
        acc_v[...] = acc
        pltpu.sync_copy(acc_v, out.at[wid])

    return tag_kernel


def _det_body(det_scale, preds_ref, heat_ref, mask_ref, out_ref):
    b = pl.program_id(0)
    d = preds_ref[...]
    h = heat_ref[...]
    m = mask_ref[...]
    psum = jnp.sum((d - h[:, None]) ** 2 * m[:, None, None])

    @pl.when(b == 0)
    def _():
        out_ref[0, 0] = 0.0

    out_ref[0, 0] = out_ref[0, 0] + psum * det_scale


def kernel(preds, masks, keypoints, gt_tags, heatmaps):
    loss_weights = (0.001, 1.0)
    B, nstack, chans, H, W = preds.shape
    n_parts = heatmaps.shape[1]
    tag_dim = gt_tags.shape[1]
    P, K = keypoints.shape[1], keypoints.shape[2]
    pk = P * K
    n_chunks = -(-pk // 128)
    pk_pad = n_chunks * 128
    n_workers = 32
    hw = H * W

    preds_flat = preds.reshape(-1)
    idx = keypoints[..., 0].astype(jnp.int32).reshape(B, pk)
    vis = keypoints[..., 1].astype(jnp.float32).reshape(B, pk)
    gt = gt_tags.astype(jnp.float32).reshape(B, pk)
    pad = ((0, 0), (0, pk_pad - pk))
    idx = jnp.pad(idx, pad).reshape(B, n_chunks, 128)
    vis = jnp.pad(vis, pad).reshape(B, n_chunks, 128)
    gt = jnp.pad(gt, pad).reshape(B, n_chunks, 128)

    tag_kernel = _make_tag_kernel(n_workers, n_chunks, chans, n_parts, hw)
    partials = tag_kernel(preds_flat, idx, gt, vis)

    tag_scale = loss_weights[0] / (B * nstack * tag_dim)
    det_scale = loss_weights[1] / (B * nstack * n_parts * H * W)
    out = pl.pallas_call(
        functools.partial(_det_body, det_scale),
        grid=(B // 4,),
        in_specs=[
            pl.BlockSpec((4, nstack, n_parts, H, W), lambda b: (b, 0, 0, 0, 0)),
            pl.BlockSpec((4, n_parts, H, W), lambda b: (b, 0, 0, 0)),
            pl.BlockSpec((4, H, W), lambda b: (b, 0, 0)),
        ],
        out_specs=pl.BlockSpec(memory_space=pltpu.SMEM),
        out_shape=jax.ShapeDtypeStruct((1, 1), jnp.float32),
        compiler_params=pltpu.CompilerParams(
            dimension_semantics=("arbitrary",)),
    )(preds, heatmaps, masks)
    return out[0, 0] + jnp.sum(partials) * tag_scale


# speedup vs baseline: 1.0215x; 1.0215x over previous
import functools

import jax
import jax.numpy as jnp
from jax import lax
from jax.experimental import pallas as pl
from jax.experimental.pallas import tpu as pltpu
from jax.experimental.pallas import tpu_sc as plsc

_LANES = 16


def _make_tag_kernel(n_workers, n_chunks, chans, n_parts, hw):
    mesh = plsc.VectorSubcoreMesh(core_axis_name="c", subcore_axis_name="s")

    @functools.partial(
        pl.kernel,
        mesh=mesh,
        out_type=jax.ShapeDtypeStruct((n_workers, _LANES), jnp.float32),
        scratch_types=[
            pltpu.VMEM((n_chunks, 128), jnp.int32),
            pltpu.VMEM((2 * n_chunks, 128), jnp.int32),
            pltpu.VMEM((2 * n_chunks, 128), jnp.float32),
            pltpu.VMEM((n_chunks, 128), jnp.float32),
            pltpu.VMEM((n_chunks, 128), jnp.float32),
            pltpu.VMEM((_LANES,), jnp.float32),
            pltpu.SemaphoreType.DMA,
        ],
    )
    def tag_kernel(preds_flat, kp_idx, gt, vis, out,
                   idx_v, addr_v, vals_v, gt_v, vis_v, acc_v, sem):
        wid = lax.axis_index("s") * 2 + lax.axis_index("c")
        b = wid // 2
        pltpu.sync_copy(kp_idx.at[b], idx_v)
        pltpu.sync_copy(gt.at[b], gt_v)
        pltpu.sync_copy(vis.at[b], vis_v)
        for t in range(2):
            j = wid * 2 + t
            base = (j * chans + n_parts) * hw
            for c in range(n_chunks):
                for i in range(128 // _LANES):
                    sl = pl.ds(i * _LANES, _LANES)
                    addr_v[t * n_chunks + c, sl] = idx_v[c, sl] + base
        copies = [
            pltpu.async_copy(preds_flat.at[addr_v.at[r]], vals_v.at[r], sem)
            for r in range(2 * n_chunks)
        ]
        for cp in copies:
            cp.wait()
        acc = jnp.zeros((_LANES,), jnp.float32)
        for t in range(2):
            for c in range(n_chunks):
                for i in range(128 // _LANES):
                    sl = pl.ds(i * _LANES, _LANES)
                    d = vals_v[t * n_chunks + c, sl] - gt_v[c, sl]
                    acc = acc + d * d * vis_v[c, sl]
        acc_v[...] = acc
        pltpu.sync_copy(acc_v, out.at[wid])

    return tag_kernel


def _det_body(det_scale, preds_ref, heat_ref, mask_ref, out_ref):
    b = pl.program_id(0)
    d = preds_ref[...]
    h = heat_ref[...]
    m = mask_ref[...]
    psum = jnp.sum((d - h[:, None]) ** 2 * m[:, None, None])

    @pl.when(b == 0)
    def _():
        out_ref[0, 0] = 0.0

    out_ref[0, 0] = out_ref[0, 0] + psum * det_scale


def kernel(preds, masks, keypoints, gt_tags, heatmaps):
    loss_weights = (0.001, 1.0)
    B, nstack, chans, H, W = preds.shape
    n_parts = heatmaps.shape[1]
    tag_dim = gt_tags.shape[1]
    P, K = keypoints.shape[1], keypoints.shape[2]
    pk = P * K
    n_chunks = -(-pk // 128)
    pk_pad = n_chunks * 128
    n_workers = 32
    hw = H * W

    preds_flat = preds.reshape(-1)
    idx = keypoints[..., 0].astype(jnp.int32).reshape(B, pk)
    vis = keypoints[..., 1].astype(jnp.float32).reshape(B, pk)
    gt = gt_tags.astype(jnp.float32).reshape(B, pk)
    pad = ((0, 0), (0, pk_pad - pk))
    idx = jnp.pad(idx, pad).reshape(B, n_chunks, 128)
    vis = jnp.pad(vis, pad).reshape(B, n_chunks, 128)
    gt = jnp.pad(gt, pad).reshape(B, n_chunks, 128)

    tag_kernel = _make_tag_kernel(n_workers, n_chunks, chans, n_parts, hw)
    partials = tag_kernel(preds_flat, idx, gt, vis)

    tag_scale = loss_weights[0] / (B * nstack * tag_dim)
    det_scale = loss_weights[1] / (B * nstack * n_parts * H * W)
    out = pl.pallas_call(
        functools.partial(_det_body, det_scale),
        grid=(B // 2,),
        in_specs=[
            pl.BlockSpec((2, nstack, n_parts, H, W), lambda b: (b, 0, 0, 0, 0)),
            pl.BlockSpec((2, n_parts, H, W), lambda b: (b, 0, 0, 0)),
            pl.BlockSpec((2, H, W), lambda b: (b, 0, 0)),
        ],
        out_specs=pl.BlockSpec(memory_space=pltpu.SMEM),
        out_shape=jax.ShapeDtypeStruct((1, 1), jnp.float32),
        compiler_params=pltpu.CompilerParams(
            dimension_semantics=("arbitrary",)),
    )(preds, heatmaps, masks)
    return out[0, 0] + jnp.sum(partials) * tag_scale
